# Initial kernel scaffold; baseline (speedup 1.0000x reference)
import jax
import jax.numpy as jnp
from jax.experimental import pallas as pl


def kernel(uids, iids, pos, neg, E_u_0, E_i_0, adj_rows, adj_cols, adj_vals, u_mul_s, v_mul_s, ut, vt):
    raise NotImplementedError("write your pallas kernel here")


# speedup vs baseline: 3.4720x; 3.4720x over previous
import functools

import jax
import jax.numpy as jnp
from jax import lax
from jax.experimental import pallas as pl
from jax.experimental.pallas import tpu as pltpu
from jax.experimental.pallas import tpu_sc as plsc

N_U = 10000
N_I = 10000
D = 128
E = 320000
R = 16
B = 4096
TEMP = 0.2
LAMBDA1 = 0.2
LAMBDA2 = 1e-07

NC = 2
NS = 16
K = 128
TPE = 20096
NCHUNK = TPE // K
EP = NS * TPE
RPT = 624
TAIL = N_U - NS * RPT

def _spmm_body(tab_u, tab_i, rows_h, cols_h, vals_h, zeros_h,
               zu_out, zi_out, acc, gidx, sidx, vv, rbuf, sem):
  c = lax.axis_index("c")
  s = lax.axis_index("s")
  row_lo = s * RPT
  pltpu.sync_copy(zeros_h.at[pl.ds(row_lo, RPT), :],
                  acc.at[pl.ds(row_lo, RPT), :])

  @pl.when(s == 0)
  def _():
    pltpu.sync_copy(zeros_h.at[pl.ds(NS * RPT, TAIL), :],
                    acc.at[pl.ds(NS * RPT, TAIL), :])

  plsc.subcore_barrier()

  def side(tab, gidx_h, sidx_h, out_h):
    def chunk(ci, carry):
      base = s * TPE + ci * K
      pltpu.sync_copy(gidx_h.at[pl.ds(base, K)], gidx)
      pltpu.sync_copy(sidx_h.at[pl.ds(base, K)], sidx)
      pltpu.sync_copy(vals_h.at[pl.ds(base, K)], vv)
      pltpu.async_copy(tab.at[gidx], rbuf, sem).wait()

      def scale(g, carry2):
        vvv = vv[pl.ds(g * 16, 16)]
        for e16 in range(16):
          e = g * 16 + e16
          sv = jnp.broadcast_to(vvv[e16], (16,))
          for db in range(D // 16):
            rbuf[e, pl.ds(db * 16, 16)] = rbuf[e, pl.ds(db * 16, 16)] * sv
        return carry2

      lax.fori_loop(0, K // 16, scale, 0)
      pltpu.sync_copy(rbuf, acc.at[sidx], add=True)
      return carry

    lax.fori_loop(0, NCHUNK, chunk, 0)
    plsc.subcore_barrier()
    pltpu.sync_copy(acc.at[pl.ds(row_lo, RPT), :],
                    out_h.at[pl.ds(row_lo, RPT), :])

    @pl.when(s == 0)
    def _():
      pltpu.sync_copy(acc.at[pl.ds(NS * RPT, TAIL), :],
                      out_h.at[pl.ds(NS * RPT, TAIL), :])

  @pl.when(c == 0)
  def _():
    side(tab_i, cols_h, rows_h, zu_out)

  @pl.when(c == 1)
  def _():
    side(tab_u, rows_h, cols_h, zi_out)


@functools.lru_cache(maxsize=1)
def _sc_kernels():
  mesh = plsc.VectorSubcoreMesh(
      core_axis_name="c", subcore_axis_name="s", num_cores=NC, num_subcores=NS)
  spmm = functools.partial(
      pl.kernel,
      out_type=(jax.ShapeDtypeStruct((N_U, D), jnp.float32),
                jax.ShapeDtypeStruct((N_I, D), jnp.float32)),
      mesh=mesh,
      scratch_types=[
          pltpu.VMEM_SHARED((N_U, D), jnp.float32),
          pltpu.VMEM((K,), jnp.int32),
          pltpu.VMEM((K,), jnp.int32),
          pltpu.VMEM((K,), jnp.float32),
          pltpu.VMEM((K, D), jnp.float32),
          pltpu.SemaphoreType.DMA,
      ],
  )(_spmm_body)
  gather = functools.partial(
      pl.kernel,
      out_type=(jax.ShapeDtypeStruct((B, D), jnp.float32),
                jax.ShapeDtypeStruct((B, D), jnp.float32),
                jax.ShapeDtypeStruct((B, D), jnp.float32),
                jax.ShapeDtypeStruct((B, D), jnp.float32),
                jax.ShapeDtypeStruct((2 * B, D), jnp.float32),
                jax.ShapeDtypeStruct((2 * B, D), jnp.float32),
                jax.ShapeDtypeStruct((2 * B, D), jnp.float32),
                jax.ShapeDtypeStruct((2 * B, D), jnp.float32)),
      mesh=mesh,
      scratch_types=[
          pltpu.VMEM((K,), jnp.int32),
          pltpu.VMEM((K, D), jnp.float32),
          pltpu.SemaphoreType.DMA,
      ],
  )(_gather_body)
  return spmm, gather


def _gather_body(eu0, zu1, zu2, ums, ei0, zi1, zi2, vms, uids_h, iids_h,
                 o_eu0u, o_zu1u, o_zu2u, o_umsu, o_ei0i, o_zi1i, o_zi2i,
                 o_vmsi, idx, rbuf, sem):
  c = lax.axis_index("c")
  s = lax.axis_index("s")
  wid = s * NC + c

  def grab(tab, out_h, base):
    pltpu.async_copy(tab.at[idx], rbuf, sem).wait()
    pltpu.sync_copy(rbuf, out_h.at[pl.ds(base, K), :])

  ub = wid * K
  pltpu.sync_copy(uids_h.at[pl.ds(ub, K)], idx)
  grab(eu0, o_eu0u, ub)
  grab(zu1, o_zu1u, ub)
  grab(zu2, o_zu2u, ub)
  grab(ums, o_umsu, ub)
  for half in range(2):
    ib = wid * 2 * K + half * K
    pltpu.sync_copy(iids_h.at[pl.ds(ib, K)], idx)
    grab(ei0, o_ei0i, ib)
    grab(zi1, o_zi1i, ib)
    grab(zi2, o_zi2i, ib)
    grab(vms, o_vmsi, ib)


JBLK = 1000
NJ = N_U // JBLK


def _tca_body(vtt_b, ei0_b, zi1_b, utt_b, eu0_b, zu1_b, wu_ref, wi_ref,
              reg_ref):
  k = pl.program_id(0)
  pi = ei0_b[...] + zi1_b[...]
  pu = eu0_b[...] + zu1_b[...]
  cdims = (((0,), (0,)), ((), ()))
  vtt = vtt_b[pl.ds(k * JBLK, JBLK), :]
  utt = utt_b[pl.ds(k * JBLK, JBLK), :]
  wu = lax.dot_general(vtt, pi, cdims, preferred_element_type=jnp.float32)
  wi = lax.dot_general(utt, pu, cdims, preferred_element_type=jnp.float32)
  reg = jnp.sum(eu0_b[...] * eu0_b[...]) + jnp.sum(ei0_b[...] * ei0_b[...])

  @pl.when(k == 0)
  def _():
    wu_ref[...] = wu
    wi_ref[...] = wi
    reg_ref[0, 0] = reg

  @pl.when(k > 0)
  def _():
    wu_ref[...] += wu
    wi_ref[...] += wi
    reg_ref[0, 0] += reg


def _tca(vtt, e_i0, z_i1, utt, e_u0, z_u1):
  return pl.pallas_call(
      _tca_body,
      grid=(NJ,),
      in_specs=[
          pl.BlockSpec((N_I, R), lambda k: (0, 0)),
          pl.BlockSpec((JBLK, D), lambda k: (k, 0)),
          pl.BlockSpec((JBLK, D), lambda k: (k, 0)),
          pl.BlockSpec((N_U, R), lambda k: (0, 0)),
          pl.BlockSpec((JBLK, D), lambda k: (k, 0)),
          pl.BlockSpec((JBLK, D), lambda k: (k, 0)),
      ],
      out_specs=[
          pl.BlockSpec((R, D), lambda k: (0, 0)),
          pl.BlockSpec((R, D), lambda k: (0, 0)),
          pl.BlockSpec((1, 1), lambda k: (0, 0),
                       memory_space=pltpu.SMEM),
      ],
      out_shape=[
          jax.ShapeDtypeStruct((R, D), jnp.float32),
          jax.ShapeDtypeStruct((R, D), jnp.float32),
          jax.ShapeDtypeStruct((1, 1), jnp.float32),
      ],
  )(vtt, e_i0, z_i1, utt, e_u0, z_u1)


INV_T = 5.0


def _tcb_body(eu0_b, zu1_b, zu2_b, ei0_b, zi1_b, zi2_b,
              wu, wi, regsum,
              eu0_u, zu1_u, zu2_u, ums_u, ei0_i, zi1_i, zi2_i, vms_i,
              loss_ref, lossr_ref, losss_ref,
              gu_rows, gi_rows, acc_u, acc_i):
  j = pl.program_id(0)

  @pl.when(j == 0)
  def _():
    gu_rows[...] = eu0_u[...] + jnp.dot(ums_u[...][:, :R], wu[...],
                                        preferred_element_type=jnp.float32)
    gi_rows[...] = ei0_i[...] + jnp.dot(vms_i[...][:, :R], wi[...],
                                        preferred_element_type=jnp.float32)
    acc_u[...] = jnp.zeros_like(acc_u)
    acc_i[...] = jnp.zeros_like(acc_i)

  e_u_b = eu0_b[...] + zu1_b[...] + zu2_b[...]
  e_i_b = ei0_b[...] + zi1_b[...] + zi2_b[...]
  su = lax.dot_general(gu_rows[...], e_u_b, (((1,), (1,)), ((), ())),
                       preferred_element_type=jnp.float32)
  si = lax.dot_general(gi_rows[...], e_i_b, (((1,), (1,)), ((), ())),
                       preferred_element_type=jnp.float32)
  acc_u[...] += jnp.sum(jnp.exp(su * INV_T), axis=1, keepdims=True)
  acc_i[...] += jnp.sum(jnp.exp(si * INV_T), axis=1, keepdims=True)

  @pl.when(j == NJ - 1)
  def _():
    neg_score = (jnp.mean(jnp.log(acc_u[...] + 1e-08))
                 + jnp.mean(jnp.log(acc_i[...] + 1e-08)))
    eu_rows = eu0_u[...] + zu1_u[...] + zu2_u[...]
    ei_rows = ei0_i[...] + zi1_i[...] + zi2_i[...]
    pos_score = (
        jnp.mean(jnp.clip(jnp.sum(gu_rows[...] * eu_rows, axis=1) * INV_T,
                          -5.0, 5.0))
        + jnp.mean(jnp.clip(jnp.sum(gi_rows[...] * ei_rows, axis=1) * INV_T,
                            -5.0, 5.0)))
    loss_s = neg_score - pos_score
    pos_sc = jnp.sum(eu_rows * ei_rows[:B], axis=1)
    neg_sc = jnp.sum(eu_rows * ei_rows[B:], axis=1)
    loss_r = -jnp.mean(jnp.log(jax.nn.sigmoid(pos_sc - neg_sc)))
    reg = regsum[0, 0] * LAMBDA2
    lossr_ref[0, 0] = loss_r
    losss_ref[0, 0] = LAMBDA1 * loss_s
    loss_ref[0, 0] = loss_r + LAMBDA1 * loss_s + reg


def _tcb(e_u0, z_u1, z_u2, e_i0, z_i1, z_i2, wu, wi, regsum,
         eu0_u, zu1_u, zu2_u, ums_u, ei0_i, zi1_i, zi2_i, vms_i):
  full = lambda shape: pl.BlockSpec(shape, lambda j: (0, 0))
  blk = pl.BlockSpec((JBLK, D), lambda j: (j, 0))
  return pl.pallas_call(
      _tcb_body,
      grid=(NJ,),
      in_specs=[
          blk, blk, blk, blk, blk, blk,
          full((R, D)), full((R, D)),
          pl.BlockSpec((1, 1), lambda j: (0, 0), memory_space=pltpu.SMEM),
          full((B, D)), full((B, D)), full((B, D)), full((B, D)),
          full((2 * B, D)), full((2 * B, D)), full((2 * B, D)),
          full((2 * B, D)),
      ],
      out_specs=[
          pl.BlockSpec((1, 1), lambda j: (0, 0), memory_space=pltpu.SMEM),
          pl.BlockSpec((1, 1), lambda j: (0, 0), memory_space=pltpu.SMEM),
          pl.BlockSpec((1, 1), lambda j: (0, 0), memory_space=pltpu.SMEM),
      ],
      out_shape=[
          jax.ShapeDtypeStruct((1, 1), jnp.float32),
          jax.ShapeDtypeStruct((1, 1), jnp.float32),
          jax.ShapeDtypeStruct((1, 1), jnp.float32),
      ],
      scratch_shapes=[
          pltpu.VMEM((B, D), jnp.float32),
          pltpu.VMEM((2 * B, D), jnp.float32),
          pltpu.VMEM((B, 1), jnp.float32),
          pltpu.VMEM((2 * B, 1), jnp.float32),
      ],
  )(e_u0, z_u1, z_u2, e_i0, z_i1, z_i2, wu, wi, regsum,
    eu0_u, zu1_u, zu2_u, ums_u, ei0_i, zi1_i, zi2_i, vms_i)


def kernel(uids, iids, pos, neg, E_u_0, E_i_0, adj_rows, adj_cols, adj_vals,
           u_mul_s, v_mul_s, ut, vt):
  pad = EP - E
  rows_p = jnp.concatenate([adj_rows, jnp.zeros((pad,), jnp.int32)])
  cols_p = jnp.concatenate([adj_cols, jnp.zeros((pad,), jnp.int32)])
  vals_p = jnp.concatenate([adj_vals, jnp.zeros((pad,), jnp.float32)])
  zeros = jnp.zeros((N_U, D), jnp.float32)
  ums_p = jnp.pad(u_mul_s, ((0, 0), (0, D - R)))
  vms_p = jnp.pad(v_mul_s, ((0, 0), (0, D - R)))

  spmm, gather = _sc_kernels()
  z_u1, z_i1 = spmm(E_u_0, E_i_0, rows_p, cols_p, vals_p, zeros)
  z_u2, z_i2 = spmm(z_u1, z_i1, rows_p, cols_p, vals_p, zeros)

  wu, wi, regsum = _tca(vt.T, E_i_0, z_i1, ut.T, E_u_0, z_u1)

  (eu0_u, zu1_u, zu2_u, ums_u, ei0_i, zi1_i, zi2_i, vms_i) = gather(
      E_u_0, z_u1, z_u2, ums_p, E_i_0, z_i1, z_i2, vms_p, uids, iids)

  loss, loss_r, loss_s = _tcb(
      E_u_0, z_u1, z_u2, E_i_0, z_i1, z_i2, wu, wi, regsum,
      eu0_u, zu1_u, zu2_u, ums_u, ei0_i, zi1_i, zi2_i, vms_i)

  return (loss[0, 0], loss_r[0, 0], loss_s[0, 0])


# speedup vs baseline: 3.6137x; 1.0408x over previous
import functools

import jax
import jax.numpy as jnp
from jax import lax
from jax.experimental import pallas as pl
from jax.experimental.pallas import tpu as pltpu
from jax.experimental.pallas import tpu_sc as plsc

N_U = 10000
N_I = 10000
D = 128
E = 320000
R = 16
B = 4096
TEMP = 0.2
LAMBDA1 = 0.2
LAMBDA2 = 1e-07

NC = 2
NS = 16
K = 128
SB = 8
TPE = 20480
NCHUNK = TPE // K
NSUP = NCHUNK // SB
EP = NS * TPE
RPT = 624
TAIL = N_U - NS * RPT

def _spmm_body(tab_u, tab_i, rows_h, cols_h, vals_h, zeros_h,
               zu_out, zi_out, acc, gidx_a, sidx_a, vv_a, rbuf0, rbuf1,
               sem0, sem1):
  c = lax.axis_index("c")
  s = lax.axis_index("s")
  row_lo = s * RPT
  pltpu.sync_copy(zeros_h.at[pl.ds(row_lo, RPT), :],
                  acc.at[pl.ds(row_lo, RPT), :])

  @pl.when(s == 0)
  def _():
    pltpu.sync_copy(zeros_h.at[pl.ds(NS * RPT, TAIL), :],
                    acc.at[pl.ds(NS * RPT, TAIL), :])

  plsc.subcore_barrier()

  def side(tab, gidx_h, sidx_h, out_h):
    def scale_scatter(ci, rbuf):
      def scale(g, carry2):
        vvv = vv_a[ci, pl.ds(g * 16, 16)]
        for e16 in range(16):
          e = g * 16 + e16
          sv = jnp.broadcast_to(vvv[e16], (16,))
          for db in range(D // 16):
            rbuf[e, pl.ds(db * 16, 16)] = rbuf[e, pl.ds(db * 16, 16)] * sv
        return carry2

      lax.fori_loop(0, K // 16, scale, 0)
      pltpu.sync_copy(rbuf, acc.at[sidx_a.at[ci]], add=True)

    def superchunk(sc, carry):
      pltpu.sync_copy(gidx_h.at[s, pl.ds(sc * SB, SB)], gidx_a)
      pltpu.sync_copy(sidx_h.at[s, pl.ds(sc * SB, SB)], sidx_a)
      pltpu.sync_copy(vals_h.at[s, pl.ds(sc * SB, SB)], vv_a)
      pltpu.async_copy(tab.at[gidx_a.at[0]], rbuf0, sem0)
      for b in range(SB // 2):
        ci0 = 2 * b
        pltpu.make_async_copy(tab.at[gidx_a.at[ci0]], rbuf0, sem0).wait()
        pltpu.async_copy(tab.at[gidx_a.at[ci0 + 1]], rbuf1, sem1)
        scale_scatter(ci0, rbuf0)
        pltpu.make_async_copy(tab.at[gidx_a.at[ci0 + 1]], rbuf1, sem1).wait()
        if ci0 + 2 < SB:
          pltpu.async_copy(tab.at[gidx_a.at[ci0 + 2]], rbuf0, sem0)
        scale_scatter(ci0 + 1, rbuf1)
      return carry

    lax.fori_loop(0, NSUP, superchunk, 0)
    plsc.subcore_barrier()
    pltpu.sync_copy(acc.at[pl.ds(row_lo, RPT), :],
                    out_h.at[pl.ds(row_lo, RPT), :])

    @pl.when(s == 0)
    def _():
      pltpu.sync_copy(acc.at[pl.ds(NS * RPT, TAIL), :],
                      out_h.at[pl.ds(NS * RPT, TAIL), :])

  @pl.when(c == 0)
  def _():
    side(tab_i, cols_h, rows_h, zu_out)

  @pl.when(c == 1)
  def _():
    side(tab_u, rows_h, cols_h, zi_out)


@functools.lru_cache(maxsize=1)
def _sc_kernels():
  mesh = plsc.VectorSubcoreMesh(
      core_axis_name="c", subcore_axis_name="s", num_cores=NC, num_subcores=NS)
  spmm = functools.partial(
      pl.kernel,
      out_type=(jax.ShapeDtypeStruct((N_U, D), jnp.float32),
                jax.ShapeDtypeStruct((N_I, D), jnp.float32)),
      mesh=mesh,
      scratch_types=[
          pltpu.VMEM_SHARED((N_U, D), jnp.float32),
          pltpu.VMEM((SB, K), jnp.int32),
          pltpu.VMEM((SB, K), jnp.int32),
          pltpu.VMEM((SB, K), jnp.float32),
          pltpu.VMEM((K, D), jnp.float32),
          pltpu.VMEM((K, D), jnp.float32),
          pltpu.SemaphoreType.DMA,
          pltpu.SemaphoreType.DMA,
      ],
  )(_spmm_body)
  gather = functools.partial(
      pl.kernel,
      out_type=(jax.ShapeDtypeStruct((B, D), jnp.float32),
                jax.ShapeDtypeStruct((B, D), jnp.float32),
                jax.ShapeDtypeStruct((B, D), jnp.float32),
                jax.ShapeDtypeStruct((B, D), jnp.float32),
                jax.ShapeDtypeStruct((2 * B, D), jnp.float32),
                jax.ShapeDtypeStruct((2 * B, D), jnp.float32),
                jax.ShapeDtypeStruct((2 * B, D), jnp.float32),
                jax.ShapeDtypeStruct((2 * B, D), jnp.float32)),
      mesh=mesh,
      scratch_types=[
          pltpu.VMEM((K,), jnp.int32),
          pltpu.VMEM((K, D), jnp.float32),
          pltpu.SemaphoreType.DMA,
      ],
  )(_gather_body)
  return spmm, gather


def _gather_body(eu0, zu1, zu2, ums, ei0, zi1, zi2, vms, uids_h, iids_h,
                 o_eu0u, o_zu1u, o_zu2u, o_umsu, o_ei0i, o_zi1i, o_zi2i,
                 o_vmsi, idx, rbuf, sem):
  c = lax.axis_index("c")
  s = lax.axis_index("s")
  wid = s * NC + c

  def grab(tab, out_h, base):
    pltpu.async_copy(tab.at[idx], rbuf, sem).wait()
    pltpu.sync_copy(rbuf, out_h.at[pl.ds(base, K), :])

  ub = wid * K
  pltpu.sync_copy(uids_h.at[pl.ds(ub, K)], idx)
  grab(eu0, o_eu0u, ub)
  grab(zu1, o_zu1u, ub)
  grab(zu2, o_zu2u, ub)
  grab(ums, o_umsu, ub)
  for half in range(2):
    ib = wid * 2 * K + half * K
    pltpu.sync_copy(iids_h.at[pl.ds(ib, K)], idx)
    grab(ei0, o_ei0i, ib)
    grab(zi1, o_zi1i, ib)
    grab(zi2, o_zi2i, ib)
    grab(vms, o_vmsi, ib)


JBLK = 1000
NJ = N_U // JBLK


def _tca_body(vtt_b, ei0_b, zi1_b, utt_b, eu0_b, zu1_b, wu_ref, wi_ref,
              reg_ref):
  k = pl.program_id(0)
  pi = ei0_b[...] + zi1_b[...]
  pu = eu0_b[...] + zu1_b[...]
  cdims = (((0,), (0,)), ((), ()))
  vtt = vtt_b[pl.ds(k * JBLK, JBLK), :]
  utt = utt_b[pl.ds(k * JBLK, JBLK), :]
  wu = lax.dot_general(vtt, pi, cdims, preferred_element_type=jnp.float32)
  wi = lax.dot_general(utt, pu, cdims, preferred_element_type=jnp.float32)
  reg = jnp.sum(eu0_b[...] * eu0_b[...]) + jnp.sum(ei0_b[...] * ei0_b[...])

  @pl.when(k == 0)
  def _():
    wu_ref[...] = wu
    wi_ref[...] = wi
    reg_ref[0, 0] = reg

  @pl.when(k > 0)
  def _():
    wu_ref[...] += wu
    wi_ref[...] += wi
    reg_ref[0, 0] += reg


def _tca(vtt, e_i0, z_i1, utt, e_u0, z_u1):
  return pl.pallas_call(
      _tca_body,
      grid=(NJ,),
      in_specs=[
          pl.BlockSpec((N_I, R), lambda k: (0, 0)),
          pl.BlockSpec((JBLK, D), lambda k: (k, 0)),
          pl.BlockSpec((JBLK, D), lambda k: (k, 0)),
          pl.BlockSpec((N_U, R), lambda k: (0, 0)),
          pl.BlockSpec((JBLK, D), lambda k: (k, 0)),
          pl.BlockSpec((JBLK, D), lambda k: (k, 0)),
      ],
      out_specs=[
          pl.BlockSpec((R, D), lambda k: (0, 0)),
          pl.BlockSpec((R, D), lambda k: (0, 0)),
          pl.BlockSpec((1, 1), lambda k: (0, 0),
                       memory_space=pltpu.SMEM),
      ],
      out_shape=[
          jax.ShapeDtypeStruct((R, D), jnp.float32),
          jax.ShapeDtypeStruct((R, D), jnp.float32),
          jax.ShapeDtypeStruct((1, 1), jnp.float32),
      ],
  )(vtt, e_i0, z_i1, utt, e_u0, z_u1)


INV_T = 5.0


def _tcb_body(eu0_b, zu1_b, zu2_b, ei0_b, zi1_b, zi2_b,
              wu, wi, regsum,
              eu0_u, zu1_u, zu2_u, ums_u, ei0_i, zi1_i, zi2_i, vms_i,
              loss_ref, lossr_ref, losss_ref,
              gu_rows, gi_rows, acc_u, acc_i):
  j = pl.program_id(0)

  @pl.when(j == 0)
  def _():
    gu_rows[...] = eu0_u[...] + jnp.dot(ums_u[...][:, :R], wu[...],
                                        preferred_element_type=jnp.float32)
    gi_rows[...] = ei0_i[...] + jnp.dot(vms_i[...][:, :R], wi[...],
                                        preferred_element_type=jnp.float32)
    acc_u[...] = jnp.zeros_like(acc_u)
    acc_i[...] = jnp.zeros_like(acc_i)

  e_u_b = eu0_b[...] + zu1_b[...] + zu2_b[...]
  e_i_b = ei0_b[...] + zi1_b[...] + zi2_b[...]
  su = lax.dot_general(gu_rows[...], e_u_b, (((1,), (1,)), ((), ())),
                       preferred_element_type=jnp.float32)
  si = lax.dot_general(gi_rows[...], e_i_b, (((1,), (1,)), ((), ())),
                       preferred_element_type=jnp.float32)
  acc_u[...] += jnp.sum(jnp.exp(su * INV_T), axis=1, keepdims=True)
  acc_i[...] += jnp.sum(jnp.exp(si * INV_T), axis=1, keepdims=True)

  @pl.when(j == NJ - 1)
  def _():
    neg_score = (jnp.mean(jnp.log(acc_u[...] + 1e-08))
                 + jnp.mean(jnp.log(acc_i[...] + 1e-08)))
    eu_rows = eu0_u[...] + zu1_u[...] + zu2_u[...]
    ei_rows = ei0_i[...] + zi1_i[...] + zi2_i[...]
    pos_score = (
        jnp.mean(jnp.clip(jnp.sum(gu_rows[...] * eu_rows, axis=1) * INV_T,
                          -5.0, 5.0))
        + jnp.mean(jnp.clip(jnp.sum(gi_rows[...] * ei_rows, axis=1) * INV_T,
                            -5.0, 5.0)))
    loss_s = neg_score - pos_score
    pos_sc = jnp.sum(eu_rows * ei_rows[:B], axis=1)
    neg_sc = jnp.sum(eu_rows * ei_rows[B:], axis=1)
    loss_r = -jnp.mean(jnp.log(jax.nn.sigmoid(pos_sc - neg_sc)))
    reg = regsum[0, 0] * LAMBDA2
    lossr_ref[0, 0] = loss_r
    losss_ref[0, 0] = LAMBDA1 * loss_s
    loss_ref[0, 0] = loss_r + LAMBDA1 * loss_s + reg


def _tcb(e_u0, z_u1, z_u2, e_i0, z_i1, z_i2, wu, wi, regsum,
         eu0_u, zu1_u, zu2_u, ums_u, ei0_i, zi1_i, zi2_i, vms_i):
  full = lambda shape: pl.BlockSpec(shape, lambda j: (0, 0))
  blk = pl.BlockSpec((JBLK, D), lambda j: (j, 0))
  return pl.pallas_call(
      _tcb_body,
      grid=(NJ,),
      in_specs=[
          blk, blk, blk, blk, blk, blk,
          full((R, D)), full((R, D)),
          pl.BlockSpec((1, 1), lambda j: (0, 0), memory_space=pltpu.SMEM),
          full((B, D)), full((B, D)), full((B, D)), full((B, D)),
          full((2 * B, D)), full((2 * B, D)), full((2 * B, D)),
          full((2 * B, D)),
      ],
      out_specs=[
          pl.BlockSpec((1, 1), lambda j: (0, 0), memory_space=pltpu.SMEM),
          pl.BlockSpec((1, 1), lambda j: (0, 0), memory_space=pltpu.SMEM),
          pl.BlockSpec((1, 1), lambda j: (0, 0), memory_space=pltpu.SMEM),
      ],
      out_shape=[
          jax.ShapeDtypeStruct((1, 1), jnp.float32),
          jax.ShapeDtypeStruct((1, 1), jnp.float32),
          jax.ShapeDtypeStruct((1, 1), jnp.float32),
      ],
      scratch_shapes=[
          pltpu.VMEM((B, D), jnp.float32),
          pltpu.VMEM((2 * B, D), jnp.float32),
          pltpu.VMEM((B, 1), jnp.float32),
          pltpu.VMEM((2 * B, 1), jnp.float32),
      ],
  )(e_u0, z_u1, z_u2, e_i0, z_i1, z_i2, wu, wi, regsum,
    eu0_u, zu1_u, zu2_u, ums_u, ei0_i, zi1_i, zi2_i, vms_i)


def kernel(uids, iids, pos, neg, E_u_0, E_i_0, adj_rows, adj_cols, adj_vals,
           u_mul_s, v_mul_s, ut, vt):
  pad = EP - E
  rows_p = jnp.concatenate(
      [adj_rows, jnp.zeros((pad,), jnp.int32)]).reshape(NS, NCHUNK, K)
  cols_p = jnp.concatenate(
      [adj_cols, jnp.zeros((pad,), jnp.int32)]).reshape(NS, NCHUNK, K)
  vals_p = jnp.concatenate(
      [adj_vals, jnp.zeros((pad,), jnp.float32)]).reshape(NS, NCHUNK, K)
  zeros = jnp.zeros((N_U, D), jnp.float32)
  ums_p = jnp.pad(u_mul_s, ((0, 0), (0, D - R)))
  vms_p = jnp.pad(v_mul_s, ((0, 0), (0, D - R)))

  spmm, gather = _sc_kernels()
  z_u1, z_i1 = spmm(E_u_0, E_i_0, rows_p, cols_p, vals_p, zeros)
  z_u2, z_i2 = spmm(z_u1, z_i1, rows_p, cols_p, vals_p, zeros)

  wu, wi, regsum = _tca(vt.T, E_i_0, z_i1, ut.T, E_u_0, z_u1)

  (eu0_u, zu1_u, zu2_u, ums_u, ei0_i, zi1_i, zi2_i, vms_i) = gather(
      E_u_0, z_u1, z_u2, ums_p, E_i_0, z_i1, z_i2, vms_p, uids, iids)

  loss, loss_r, loss_s = _tcb(
      E_u_0, z_u1, z_u2, E_i_0, z_i1, z_i2, wu, wi, regsum,
      eu0_u, zu1_u, zu2_u, ums_u, ei0_i, zi1_i, zi2_i, vms_i)

  return (loss[0, 0], loss_r[0, 0], loss_s[0, 0])


# speedup vs baseline: 3.6442x; 1.0085x over previous
import functools

import jax
import jax.numpy as jnp
from jax import lax
from jax.experimental import pallas as pl
from jax.experimental.pallas import tpu as pltpu
from jax.experimental.pallas import tpu_sc as plsc

N_U = 10000
N_I = 10000
D = 128
E = 320000
R = 16
B = 4096
TEMP = 0.2
LAMBDA1 = 0.2
LAMBDA2 = 1e-07

NC = 2
NS = 16
K = 128
SB = 8
TPE = 20480
NCHUNK = TPE // K
NSUP = NCHUNK // SB
EP = NS * TPE
RPT = 624
TAIL = N_U - NS * RPT

def _spmm_body(tab_u, tab_i, rows_h, cols_h, vals_h, zeros_h,
               zu_out, zi_out, acc, gidx_a, sidx_a, vv_a, rbuf0, rbuf1,
               sem0, sem1):
  c = lax.axis_index("c")
  s = lax.axis_index("s")
  row_lo = s * RPT
  pltpu.sync_copy(zeros_h.at[pl.ds(row_lo, RPT), :],
                  acc.at[pl.ds(row_lo, RPT), :])

  @pl.when(s == 0)
  def _():
    pltpu.sync_copy(zeros_h.at[pl.ds(NS * RPT, TAIL), :],
                    acc.at[pl.ds(NS * RPT, TAIL), :])

  plsc.subcore_barrier()

  def side(tab, gidx_h, sidx_h, out_h):
    def scale_scatter(ci, rbuf):
      def scale(g, carry2):
        vvv = vv_a[ci, pl.ds(g * 16, 16)]
        for e16 in range(16):
          e = g * 16 + e16
          sv = jnp.broadcast_to(vvv[e16], (16,))
          for db in range(D // 16):
            rbuf[e, pl.ds(db * 16, 16)] = rbuf[e, pl.ds(db * 16, 16)] * sv
        return carry2

      lax.fori_loop(0, K // 16, scale, 0)
      pltpu.sync_copy(rbuf, acc.at[sidx_a.at[ci]], add=True)

    bufs = (rbuf0, rbuf1)
    sems = (sem0, sem1)

    def superchunk(sc, carry):
      pltpu.sync_copy(gidx_h.at[s, pl.ds(sc * SB, SB)], gidx_a)
      pltpu.sync_copy(sidx_h.at[s, pl.ds(sc * SB, SB)], sidx_a)
      pltpu.sync_copy(vals_h.at[s, pl.ds(sc * SB, SB)], vv_a)
      pltpu.async_copy(tab.at[gidx_a.at[0]], rbuf0, sem0)
      pltpu.async_copy(tab.at[gidx_a.at[1]], rbuf1, sem1)
      for j in range(SB):
        b, sm = bufs[j % 2], sems[j % 2]
        pltpu.make_async_copy(tab.at[gidx_a.at[j]], b, sm).wait()
        scale_scatter(j, b)
        if j + 2 < SB:
          pltpu.async_copy(tab.at[gidx_a.at[j + 2]], b, sm)
      return carry

    lax.fori_loop(0, NSUP, superchunk, 0)
    plsc.subcore_barrier()
    pltpu.sync_copy(acc.at[pl.ds(row_lo, RPT), :],
                    out_h.at[pl.ds(row_lo, RPT), :])

    @pl.when(s == 0)
    def _():
      pltpu.sync_copy(acc.at[pl.ds(NS * RPT, TAIL), :],
                      out_h.at[pl.ds(NS * RPT, TAIL), :])

  @pl.when(c == 0)
  def _():
    side(tab_i, cols_h, rows_h, zu_out)

  @pl.when(c == 1)
  def _():
    side(tab_u, rows_h, cols_h, zi_out)


@functools.lru_cache(maxsize=1)
def _sc_kernels():
  mesh = plsc.VectorSubcoreMesh(
      core_axis_name="c", subcore_axis_name="s", num_cores=NC, num_subcores=NS)
  spmm = functools.partial(
      pl.kernel,
      out_type=(jax.ShapeDtypeStruct((N_U, D), jnp.float32),
                jax.ShapeDtypeStruct((N_I, D), jnp.float32)),
      mesh=mesh,
      scratch_types=[
          pltpu.VMEM_SHARED((N_U, D), jnp.float32),
          pltpu.VMEM((SB, K), jnp.int32),
          pltpu.VMEM((SB, K), jnp.int32),
          pltpu.VMEM((SB, K), jnp.float32),
          pltpu.VMEM((K, D), jnp.float32),
          pltpu.VMEM((K, D), jnp.float32),
          pltpu.SemaphoreType.DMA,
          pltpu.SemaphoreType.DMA,
      ],
  )(_spmm_body)
  gather = functools.partial(
      pl.kernel,
      out_type=(jax.ShapeDtypeStruct((B, D), jnp.float32),
                jax.ShapeDtypeStruct((B, D), jnp.float32),
                jax.ShapeDtypeStruct((B, D), jnp.float32),
                jax.ShapeDtypeStruct((B, D), jnp.float32),
                jax.ShapeDtypeStruct((2 * B, D), jnp.float32),
                jax.ShapeDtypeStruct((2 * B, D), jnp.float32),
                jax.ShapeDtypeStruct((2 * B, D), jnp.float32),
                jax.ShapeDtypeStruct((2 * B, D), jnp.float32)),
      mesh=mesh,
      scratch_types=[
          pltpu.VMEM((K,), jnp.int32),
          pltpu.VMEM((K, D), jnp.float32),
          pltpu.SemaphoreType.DMA,
      ],
  )(_gather_body)
  return spmm, gather


def _gather_body(eu0, zu1, zu2, ums, ei0, zi1, zi2, vms, uids_h, iids_h,
                 o_eu0u, o_zu1u, o_zu2u, o_umsu, o_ei0i, o_zi1i, o_zi2i,
                 o_vmsi, idx, rbuf, sem):
  c = lax.axis_index("c")
  s = lax.axis_index("s")
  wid = s * NC + c

  def grab(tab, out_h, base):
    pltpu.async_copy(tab.at[idx], rbuf, sem).wait()
    pltpu.sync_copy(rbuf, out_h.at[pl.ds(base, K), :])

  ub = wid * K
  pltpu.sync_copy(uids_h.at[pl.ds(ub, K)], idx)
  grab(eu0, o_eu0u, ub)
  grab(zu1, o_zu1u, ub)
  grab(zu2, o_zu2u, ub)
  grab(ums, o_umsu, ub)
  for half in range(2):
    ib = wid * 2 * K + half * K
    pltpu.sync_copy(iids_h.at[pl.ds(ib, K)], idx)
    grab(ei0, o_ei0i, ib)
    grab(zi1, o_zi1i, ib)
    grab(zi2, o_zi2i, ib)
    grab(vms, o_vmsi, ib)


JBLK = 1000
NJ = N_U // JBLK


def _tca_body(vtt_b, ei0_b, zi1_b, utt_b, eu0_b, zu1_b, wu_ref, wi_ref,
              reg_ref):
  k = pl.program_id(0)
  pi = ei0_b[...] + zi1_b[...]
  pu = eu0_b[...] + zu1_b[...]
  cdims = (((0,), (0,)), ((), ()))
  vtt = vtt_b[pl.ds(k * JBLK, JBLK), :]
  utt = utt_b[pl.ds(k * JBLK, JBLK), :]
  wu = lax.dot_general(vtt, pi, cdims, preferred_element_type=jnp.float32)
  wi = lax.dot_general(utt, pu, cdims, preferred_element_type=jnp.float32)
  reg = jnp.sum(eu0_b[...] * eu0_b[...]) + jnp.sum(ei0_b[...] * ei0_b[...])

  @pl.when(k == 0)
  def _():
    wu_ref[...] = wu
    wi_ref[...] = wi
    reg_ref[0, 0] = reg

  @pl.when(k > 0)
  def _():
    wu_ref[...] += wu
    wi_ref[...] += wi
    reg_ref[0, 0] += reg


def _tca(vtt, e_i0, z_i1, utt, e_u0, z_u1):
  return pl.pallas_call(
      _tca_body,
      grid=(NJ,),
      in_specs=[
          pl.BlockSpec((N_I, R), lambda k: (0, 0)),
          pl.BlockSpec((JBLK, D), lambda k: (k, 0)),
          pl.BlockSpec((JBLK, D), lambda k: (k, 0)),
          pl.BlockSpec((N_U, R), lambda k: (0, 0)),
          pl.BlockSpec((JBLK, D), lambda k: (k, 0)),
          pl.BlockSpec((JBLK, D), lambda k: (k, 0)),
      ],
      out_specs=[
          pl.BlockSpec((R, D), lambda k: (0, 0)),
          pl.BlockSpec((R, D), lambda k: (0, 0)),
          pl.BlockSpec((1, 1), lambda k: (0, 0),
                       memory_space=pltpu.SMEM),
      ],
      out_shape=[
          jax.ShapeDtypeStruct((R, D), jnp.float32),
          jax.ShapeDtypeStruct((R, D), jnp.float32),
          jax.ShapeDtypeStruct((1, 1), jnp.float32),
      ],
  )(vtt, e_i0, z_i1, utt, e_u0, z_u1)


INV_T = 5.0


def _tcb_body(eu0_b, zu1_b, zu2_b, ei0_b, zi1_b, zi2_b,
              wu, wi, regsum,
              eu0_u, zu1_u, zu2_u, ums_u, ei0_i, zi1_i, zi2_i, vms_i,
              loss_ref, lossr_ref, losss_ref,
              gu_rows, gi_rows, acc_u, acc_i):
  j = pl.program_id(0)

  @pl.when(j == 0)
  def _():
    gu_rows[...] = eu0_u[...] + jnp.dot(ums_u[...][:, :R], wu[...],
                                        preferred_element_type=jnp.float32)
    gi_rows[...] = ei0_i[...] + jnp.dot(vms_i[...][:, :R], wi[...],
                                        preferred_element_type=jnp.float32)
    acc_u[...] = jnp.zeros_like(acc_u)
    acc_i[...] = jnp.zeros_like(acc_i)

  e_u_b = eu0_b[...] + zu1_b[...] + zu2_b[...]
  e_i_b = ei0_b[...] + zi1_b[...] + zi2_b[...]
  su = lax.dot_general(gu_rows[...], e_u_b, (((1,), (1,)), ((), ())),
                       preferred_element_type=jnp.float32)
  si = lax.dot_general(gi_rows[...], e_i_b, (((1,), (1,)), ((), ())),
                       preferred_element_type=jnp.float32)
  acc_u[...] += jnp.sum(jnp.exp(su * INV_T), axis=1, keepdims=True)
  acc_i[...] += jnp.sum(jnp.exp(si * INV_T), axis=1, keepdims=True)

  @pl.when(j == NJ - 1)
  def _():
    neg_score = (jnp.mean(jnp.log(acc_u[...] + 1e-08))
                 + jnp.mean(jnp.log(acc_i[...] + 1e-08)))
    eu_rows = eu0_u[...] + zu1_u[...] + zu2_u[...]
    ei_rows = ei0_i[...] + zi1_i[...] + zi2_i[...]
    pos_score = (
        jnp.mean(jnp.clip(jnp.sum(gu_rows[...] * eu_rows, axis=1) * INV_T,
                          -5.0, 5.0))
        + jnp.mean(jnp.clip(jnp.sum(gi_rows[...] * ei_rows, axis=1) * INV_T,
                            -5.0, 5.0)))
    loss_s = neg_score - pos_score
    pos_sc = jnp.sum(eu_rows * ei_rows[:B], axis=1)
    neg_sc = jnp.sum(eu_rows * ei_rows[B:], axis=1)
    loss_r = -jnp.mean(jnp.log(jax.nn.sigmoid(pos_sc - neg_sc)))
    reg = regsum[0, 0] * LAMBDA2
    lossr_ref[0, 0] = loss_r
    losss_ref[0, 0] = LAMBDA1 * loss_s
    loss_ref[0, 0] = loss_r + LAMBDA1 * loss_s + reg


def _tcb(e_u0, z_u1, z_u2, e_i0, z_i1, z_i2, wu, wi, regsum,
         eu0_u, zu1_u, zu2_u, ums_u, ei0_i, zi1_i, zi2_i, vms_i):
  full = lambda shape: pl.BlockSpec(shape, lambda j: (0, 0))
  blk = pl.BlockSpec((JBLK, D), lambda j: (j, 0))
  return pl.pallas_call(
      _tcb_body,
      grid=(NJ,),
      in_specs=[
          blk, blk, blk, blk, blk, blk,
          full((R, D)), full((R, D)),
          pl.BlockSpec((1, 1), lambda j: (0, 0), memory_space=pltpu.SMEM),
          full((B, D)), full((B, D)), full((B, D)), full((B, D)),
          full((2 * B, D)), full((2 * B, D)), full((2 * B, D)),
          full((2 * B, D)),
      ],
      out_specs=[
          pl.BlockSpec((1, 1), lambda j: (0, 0), memory_space=pltpu.SMEM),
          pl.BlockSpec((1, 1), lambda j: (0, 0), memory_space=pltpu.SMEM),
          pl.BlockSpec((1, 1), lambda j: (0, 0), memory_space=pltpu.SMEM),
      ],
      out_shape=[
          jax.ShapeDtypeStruct((1, 1), jnp.float32),
          jax.ShapeDtypeStruct((1, 1), jnp.float32),
          jax.ShapeDtypeStruct((1, 1), jnp.float32),
      ],
      scratch_shapes=[
          pltpu.VMEM((B, D), jnp.float32),
          pltpu.VMEM((2 * B, D), jnp.float32),
          pltpu.VMEM((B, 1), jnp.float32),
          pltpu.VMEM((2 * B, 1), jnp.float32),
      ],
  )(e_u0, z_u1, z_u2, e_i0, z_i1, z_i2, wu, wi, regsum,
    eu0_u, zu1_u, zu2_u, ums_u, ei0_i, zi1_i, zi2_i, vms_i)


def kernel(uids, iids, pos, neg, E_u_0, E_i_0, adj_rows, adj_cols, adj_vals,
           u_mul_s, v_mul_s, ut, vt):
  pad = EP - E
  rows_p = jnp.concatenate(
      [adj_rows, jnp.zeros((pad,), jnp.int32)]).reshape(NS, NCHUNK, K)
  cols_p = jnp.concatenate(
      [adj_cols, jnp.zeros((pad,), jnp.int32)]).reshape(NS, NCHUNK, K)
  vals_p = jnp.concatenate(
      [adj_vals, jnp.zeros((pad,), jnp.float32)]).reshape(NS, NCHUNK, K)
  zeros = jnp.zeros((N_U, D), jnp.float32)
  ums_p = jnp.pad(u_mul_s, ((0, 0), (0, D - R)))
  vms_p = jnp.pad(v_mul_s, ((0, 0), (0, D - R)))

  spmm, gather = _sc_kernels()
  z_u1, z_i1 = spmm(E_u_0, E_i_0, rows_p, cols_p, vals_p, zeros)
  z_u2, z_i2 = spmm(z_u1, z_i1, rows_p, cols_p, vals_p, zeros)

  wu, wi, regsum = _tca(vt.T, E_i_0, z_i1, ut.T, E_u_0, z_u1)

  (eu0_u, zu1_u, zu2_u, ums_u, ei0_i, zi1_i, zi2_i, vms_i) = gather(
      E_u_0, z_u1, z_u2, ums_p, E_i_0, z_i1, z_i2, vms_p, uids, iids)

  loss, loss_r, loss_s = _tcb(
      E_u_0, z_u1, z_u2, E_i_0, z_i1, z_i2, wu, wi, regsum,
      eu0_u, zu1_u, zu2_u, ums_u, ei0_i, zi1_i, zi2_i, vms_i)

  return (loss[0, 0], loss_r[0, 0], loss_s[0, 0])
